# baseline (speedup 1.0000x reference)
import functools

import jax
import jax.numpy as jnp
from jax import lax
from jax.experimental import pallas as pl
from jax.experimental.pallas import tpu as pltpu
from jax.experimental.pallas import tpu_sc as plsc

_B, _L, _D, _OUT = 4096, 200, 32, 128
_PAD = 0
_NC, _NS = 2, 16
_NW = _NC * _NS
_BPW = _B // _NW
_C0, _C1 = 104, 96


def _sc_pool_body(x_hbm, table_hbm, acc_hbm, idx_v, rows_a, rows_b, emb_v,
                  sem_a, sem_b):
    wid = lax.axis_index("s") * _NC + lax.axis_index("c")
    base = wid * _BPW
    pltpu.sync_copy(x_hbm.at[pl.ds(base * _L, _BPW * _L)], idx_v)

    def issue(b, rows, sem):
        off = b * _L
        pltpu.async_copy(table_hbm.at[idx_v.at[pl.ds(off, _C0)]],
                         rows.at[pl.ds(0, _C0)], sem)
        pltpu.async_copy(table_hbm.at[idx_v.at[pl.ds(off + _C0, _C1)]],
                         rows.at[pl.ds(_C0, _C1)], sem)

    def drain(b, rows, sem):
        off = b * _L
        pltpu.make_async_copy(table_hbm.at[idx_v.at[pl.ds(off, _C0)]],
                              rows.at[pl.ds(0, _C0)], sem).wait()
        pltpu.make_async_copy(table_hbm.at[idx_v.at[pl.ds(off + _C0, _C1)]],
                              rows.at[pl.ds(_C0, _C1)], sem).wait()

    def accum(b, rows):
        lo = [rows[j, 0:16] for j in range(4)]
        hi = [rows[j, 16:32] for j in range(4)]
        for j in range(4, _L):
            lo[j % 4] = lo[j % 4] + rows[j, 0:16]
            hi[j % 4] = hi[j % 4] + rows[j, 16:32]
        emb_v[b, 0:16] = (lo[0] + lo[1]) + (lo[2] + lo[3])
        emb_v[b, 16:32] = (hi[0] + hi[1]) + (hi[2] + hi[3])

    issue(0, rows_a, sem_a)

    def body(i, carry):
        b0 = 2 * i
        issue(b0 + 1, rows_b, sem_b)
        drain(b0, rows_a, sem_a)
        accum(b0, rows_a)

        @pl.when(b0 + 2 < _BPW)
        def _():
            issue(b0 + 2, rows_a, sem_a)

        drain(b0 + 1, rows_b, sem_b)
        accum(b0 + 1, rows_b)
        return carry

    lax.fori_loop(0, _BPW // 2, body, 0)
    pltpu.sync_copy(emb_v, acc_hbm.at[pl.ds(base, _BPW)])


_sc_pool = pl.kernel(
    _sc_pool_body,
    out_type=jax.ShapeDtypeStruct((_B, _D), jnp.float32),
    mesh=plsc.VectorSubcoreMesh(core_axis_name="c", subcore_axis_name="s"),
    scratch_types=[
        pltpu.VMEM((_BPW * _L,), jnp.int32),
        pltpu.VMEM((_L, _D), jnp.float32),
        pltpu.VMEM((_L, _D), jnp.float32),
        pltpu.VMEM((_BPW, _D), jnp.float32),
        pltpu.SemaphoreType.DMA,
        pltpu.SemaphoreType.DMA,
    ],
    compiler_params=pltpu.CompilerParams(use_tc_tiling_on_sc=False),
)


_V = 1000000
_NCH = _V // 128
_TAIL = _V - _NCH * 128


def _sc_fmt_body(tt_hbm, tail_hbm, out_hbm, in_a, in_b, out_a, out_b, tail_v,
                 sem_ia, sem_ib, sem_oa, sem_ob):
    wid = lax.axis_index("s") * _NC + lax.axis_index("c")
    idx_base = lax.iota(jnp.int32, 16) * 32

    def issue_in(c, buf, sem):
        pltpu.async_copy(tt_hbm.at[pl.ds(0, _D), pl.ds(c * 128, 128)], buf, sem)

    def drain_in(buf, sem):
        pltpu.make_async_copy(tt_hbm.at[pl.ds(0, _D), pl.ds(0, 128)], buf,
                              sem).wait()

    def transpose(in_v, out_v):
        def dbody(d, carry):
            for g in range(8):
                v = in_v[d, 16 * g:16 * g + 16]
                plsc.store_scatter(out_v, [idx_base + (512 * g + d)], v)
            return carry
        lax.fori_loop(0, _D, dbody, 0)

    def issue_out(c, buf, sem):
        pltpu.async_copy(buf, out_hbm.at[pl.ds(c * 4096, 4096)], sem)

    def drain_out(buf, sem):
        pltpu.make_async_copy(buf, out_hbm.at[pl.ds(0, 4096)], sem).wait()

    issue_in(wid, in_a, sem_ia)

    def body(p, carry):
        c0 = wid + 64 * p
        c1 = c0 + 32

        @pl.when(c1 < _NCH)
        def _():
            issue_in(c1, in_b, sem_ib)

        @pl.when(c0 < _NCH)
        def _():
            drain_in(in_a, sem_ia)

            @pl.when(p > 0)
            def _():
                drain_out(out_a, sem_oa)

            transpose(in_a, out_a)
            issue_out(c0, out_a, sem_oa)

        @pl.when(c0 + 64 < _NCH)
        def _():
            issue_in(c0 + 64, in_a, sem_ia)

        @pl.when(c1 < _NCH)
        def _():
            drain_in(in_b, sem_ib)

            @pl.when(p > 0)
            def _():
                drain_out(out_b, sem_ob)

            transpose(in_b, out_b)
            issue_out(c1, out_b, sem_ob)

        return carry

    lax.fori_loop(0, (_NCH // 64) + 1, body, 0)
    drain_out(out_a, sem_oa)
    drain_out(out_b, sem_ob)

    @pl.when(wid == 0)
    def _():
        pltpu.sync_copy(tail_hbm, tail_v)
        pltpu.sync_copy(tail_v, out_hbm.at[pl.ds(_NCH * 4096, _TAIL * _D)])


_sc_fmt = pl.kernel(
    _sc_fmt_body,
    out_type=jax.ShapeDtypeStruct((_V * _D,), jnp.float32),
    mesh=plsc.VectorSubcoreMesh(core_axis_name="c", subcore_axis_name="s"),
    scratch_types=[
        pltpu.VMEM((_D, 128), jnp.float32),
        pltpu.VMEM((_D, 128), jnp.float32),
        pltpu.VMEM((128 * _D,), jnp.float32),
        pltpu.VMEM((128 * _D,), jnp.float32),
        pltpu.VMEM((_TAIL * _D,), jnp.float32),
        pltpu.SemaphoreType.DMA,
        pltpu.SemaphoreType.DMA,
        pltpu.SemaphoreType.DMA,
        pltpu.SemaphoreType.DMA,
    ],
    compiler_params=pltpu.CompilerParams(use_tc_tiling_on_sc=True,
                                         needs_layout_passes=False),
)


_BT = 512


def _tc_finish_body(x_ref, acc_ref, t0_ref, wt_ref, b_ref, out_ref, emb_ref):
    xb = x_ref[...]
    lens = jnp.sum((xb != _PAD).astype(jnp.float32), axis=1, keepdims=True)
    num = acc_ref[...] - (jnp.float32(_L) - lens) * t0_ref[...]
    emb = num / (lens * lens)
    emb_ref[...] = emb
    out_ref[...] = (
        jnp.dot(emb, wt_ref[...], preferred_element_type=jnp.float32)
        + b_ref[...]
    )


_tc_finish = pl.pallas_call(
    _tc_finish_body,
    grid=(_B // _BT,),
    in_specs=[
        pl.BlockSpec((_BT, _L), lambda i: (i, 0)),
        pl.BlockSpec((_BT, _D), lambda i: (i, 0)),
        pl.BlockSpec((1, _D), lambda i: (0, 0)),
        pl.BlockSpec((_D, _OUT), lambda i: (0, 0)),
        pl.BlockSpec((1, _OUT), lambda i: (0, 0)),
    ],
    out_specs=[
        pl.BlockSpec((_BT, _OUT), lambda i: (i, 0)),
        pl.BlockSpec((_BT, _D), lambda i: (i, 0)),
    ],
    out_shape=[
        jax.ShapeDtypeStruct((_B, _OUT), jnp.float32),
        jax.ShapeDtypeStruct((_B, _D), jnp.float32),
    ],
)


@jax.jit
def kernel(x, table, w, W_out, b_out):
    del w
    x = x.astype(jnp.int32)
    tail = lax.slice(table, (_NCH * 128, 0), (_V, _D)).reshape(_TAIL * _D)
    t_lin = _sc_fmt(table.T, tail)
    acc = _sc_pool(x.reshape(_B * _L), t_lin.reshape(_V, _D))
    t0 = lax.slice(table, (0, 0), (1, _D))
    out, emb = _tc_finish(x, acc, t0, W_out.T, b_out.reshape(1, _OUT))
    return (out, emb)
